# baseline (speedup 1.0000x reference)
import jax
import jax.numpy as jnp
from jax.experimental import pallas as pl

_GS = 4


def _vq_block(z_ref, e_ref, en_ref, kf_ref, w_ref, g4_ref, zq_ref, idx_ref):
    zb = z_ref[...]
    e = e_ref[...]
    en = en_ref[...]
    kf = kf_ref[...]
    w = w_ref[...]
    g4 = g4_ref[...]
    zn = jnp.sum(zb * zb, axis=-1, keepdims=True)
    dots = jnp.dot(zb, e.T, preferred_element_type=jnp.float32)
    d = zn + en[None, :] - 2.0 * dots
    m = jnp.min(d, axis=-1, keepdims=True)
    mask = (d == m).astype(jnp.float32)
    bits = jnp.dot(mask, w, preferred_element_type=jnp.float32)
    ebias = jax.lax.bitcast_convert_type(bits, jnp.int32) >> 23
    kcand = g4[None, :] + (127 - ebias)
    kcf = jnp.where(bits > 0.0, kcand.astype(jnp.float32), float(d.shape[1]))
    idxf = jnp.min(kcf, axis=-1)
    idx_ref[...] = idxf.astype(jnp.int32)
    oh = (kf[None, :] == idxf[:, None]).astype(jnp.float32)
    zq = jnp.dot(oh, e, preferred_element_type=jnp.float32)
    zq_ref[...] = zb + (zq - zb)


def kernel(z, emb_weight):
    B, T, D = z.shape
    K = emb_weight.shape[0]
    G = K // _GS
    zf = z.reshape(B * T, D)
    en = jnp.sum(emb_weight ** 2, axis=-1)
    kf = jnp.arange(K, dtype=jnp.float32)
    karange = jnp.arange(K)
    w = jnp.where(
        (karange // _GS)[:, None] == jnp.arange(G)[None, :],
        jnp.exp2(-(karange % _GS).astype(jnp.float32))[:, None],
        0.0,
    )
    g4 = (jnp.arange(G, dtype=jnp.int32) * _GS)
    TB = 1024
    grid = (B * T) // TB

    zq, idx = pl.pallas_call(
        _vq_block,
        grid=(grid,),
        in_specs=[
            pl.BlockSpec((TB, D), lambda i: (i, 0)),
            pl.BlockSpec((K, D), lambda i: (0, 0)),
            pl.BlockSpec((K,), lambda i: (0,)),
            pl.BlockSpec((K,), lambda i: (0,)),
            pl.BlockSpec((K, G), lambda i: (0, 0)),
            pl.BlockSpec((G,), lambda i: (0,)),
        ],
        out_specs=[
            pl.BlockSpec((TB, D), lambda i: (i, 0)),
            pl.BlockSpec((TB,), lambda i: (i,)),
        ],
        out_shape=[
            jax.ShapeDtypeStruct((B * T, D), jnp.float32),
            jax.ShapeDtypeStruct((B * T,), jnp.int32),
        ],
    )(zf, emb_weight, en, kf, w, g4)
    return zq.reshape(B, T, D), idx.reshape(B, T)
